# baseline (speedup 1.0000x reference)
import functools

import jax
import jax.numpy as jnp
from jax import lax
from jax.experimental import pallas as pl
from jax.experimental.pallas import tpu as pltpu
from jax.experimental.pallas import tpu_sc as plsc

_NUM_VOCAB = 100000
_MAXLEN = 200
_HID = 64
_BATCH = 4096
_SEQ = 200

_NC = 2
_NS = 16
_NW = _NC * _NS
_TOTAL = _BATCH * _SEQ
_PER_W = _TOTAL // _NW
_STEPS = _PER_W // _SEQ
_NBUF = 4
_SUBS = ((0, 128), (128, 72))


def _sc_embed(x_flat, emb_weight, pos_emb_weight):
  mesh = plsc.VectorSubcoreMesh(core_axis_name="c", subcore_axis_name="s")

  @functools.partial(
      pl.kernel,
      out_type=jax.ShapeDtypeStruct((_TOTAL, _HID), jnp.float32),
      mesh=mesh,
      compiler_params=pltpu.CompilerParams(use_tc_tiling_on_sc=False),
      scratch_types=(
          [pltpu.VMEM((_PER_W,), jnp.int32)]
          + [pltpu.VMEM((_SEQ, _HID), jnp.float32)] * _NBUF
          + [pltpu.SemaphoreType.DMA] * _NBUF
          + [pltpu.SemaphoreType.DMA] * _NBUF
      ),
  )
  def k(x_hbm, emb_hbm, pos_hbm, out_hbm, idx_v, *bufs_and_sems):
    rows = bufs_and_sems[:_NBUF]
    gsems = bufs_and_sems[_NBUF:2 * _NBUF]
    ssems = bufs_and_sems[2 * _NBUF:]
    wid = lax.axis_index("s") * _NC + lax.axis_index("c")
    wbase = wid * _PER_W
    pltpu.sync_copy(x_hbm.at[pl.ds(wbase, _PER_W)], idx_v)

    def gstart(b, s):
      row0 = s * _SEQ
      for off, n in _SUBS:
        pltpu.async_copy(
            emb_hbm.at[idx_v.at[pl.ds(row0 + off, n)]],
            rows[b].at[pl.ds(off, n)],
            gsems[b],
            add=True,
        )

    def gwait(b, s):
      row0 = s * _SEQ
      for off, n in _SUBS:
        pltpu.make_async_copy(
            emb_hbm.at[idx_v.at[pl.ds(row0 + off, n)]],
            rows[b].at[pl.ds(off, n)],
            gsems[b],
        ).wait()

    def sstart(b, s):
      pltpu.async_copy(rows[b], out_hbm.at[pl.ds(wbase + s * _SEQ, _SEQ)],
                       ssems[b])

    def swait(b, s):
      pltpu.make_async_copy(rows[b],
                            out_hbm.at[pl.ds(wbase + s * _SEQ, _SEQ)],
                            ssems[b]).wait()

    for b in range(_NBUF):
      pltpu.sync_copy(pos_hbm, rows[b])
      gstart(b, b)

    def body(t, carry):
      for b in range(_NBUF):
        s = t * _NBUF + b
        gwait(b, s)
        sstart(b, s)
        swait(b, s)
        pltpu.sync_copy(pos_hbm, rows[b])
        gstart(b, s + _NBUF)
      return carry

    lax.fori_loop(0, _STEPS // _NBUF - 1, body, 0)

    for b in range(_NBUF):
      s = _STEPS - _NBUF + b
      gwait(b, s)
      sstart(b, s)
    for b in range(_NBUF):
      swait(b, _STEPS - _NBUF + b)

  return k(x_flat, emb_weight, pos_emb_weight)


def kernel(x, emb_weight, pos_emb_weight):
  x_flat = x.reshape(-1).astype(jnp.int32)
  out = _sc_embed(x_flat, emb_weight, pos_emb_weight)
  return out.reshape(_BATCH, _SEQ, _HID)


# speedup vs baseline: 1.0106x; 1.0106x over previous
import functools

import jax
import jax.numpy as jnp
from jax import lax
from jax.experimental import pallas as pl
from jax.experimental.pallas import tpu as pltpu
from jax.experimental.pallas import tpu_sc as plsc

_NUM_VOCAB = 100000
_MAXLEN = 200
_HID = 64
_BATCH = 4096
_SEQ = 200

_NC = 2
_NS = 16
_NW = _NC * _NS
_TOTAL = _BATCH * _SEQ
_PER_W = _TOTAL // _NW
_STEPS = _PER_W // _SEQ
_NBUF = 4
_SUBS = ((0, 128), (128, 72))


def _sc_embed(x_flat, emb_weight, pos_emb_weight):
  mesh = plsc.VectorSubcoreMesh(core_axis_name="c", subcore_axis_name="s")

  @functools.partial(
      pl.kernel,
      out_type=jax.ShapeDtypeStruct((_TOTAL, _HID), jnp.float32),
      mesh=mesh,
      compiler_params=pltpu.CompilerParams(use_tc_tiling_on_sc=False),
      scratch_types=(
          [pltpu.VMEM((_PER_W,), jnp.int32)]
          + [pltpu.VMEM((_SEQ, _HID), jnp.float32)] * _NBUF
          + [pltpu.SemaphoreType.DMA] * _NBUF
          + [pltpu.SemaphoreType.DMA] * _NBUF
          + [pltpu.SemaphoreType.DMA] * _NBUF
      ),
  )
  def k(x_hbm, emb_hbm, pos_hbm, out_hbm, idx_v, *bufs_and_sems):
    rows = bufs_and_sems[:_NBUF]
    psems = bufs_and_sems[_NBUF:2 * _NBUF]
    gsems = bufs_and_sems[2 * _NBUF:3 * _NBUF]
    ssems = bufs_and_sems[3 * _NBUF:]
    wid = lax.axis_index("s") * _NC + lax.axis_index("c")
    wbase = wid * _PER_W
    pltpu.sync_copy(x_hbm.at[pl.ds(wbase, _PER_W)], idx_v)

    def pstart(b):
      pltpu.async_copy(pos_hbm, rows[b], psems[b])

    def pwait(b):
      pltpu.make_async_copy(pos_hbm, rows[b], psems[b]).wait()

    def gstart(b, s):
      row0 = s * _SEQ
      for off, n in _SUBS:
        pltpu.async_copy(
            emb_hbm.at[idx_v.at[pl.ds(row0 + off, n)]],
            rows[b].at[pl.ds(off, n)],
            gsems[b],
            add=True,
        )

    def gwait(b, s):
      row0 = s * _SEQ
      for off, n in _SUBS:
        pltpu.make_async_copy(
            emb_hbm.at[idx_v.at[pl.ds(row0 + off, n)]],
            rows[b].at[pl.ds(off, n)],
            gsems[b],
        ).wait()

    def sstart(b, s):
      pltpu.async_copy(rows[b], out_hbm.at[pl.ds(wbase + s * _SEQ, _SEQ)],
                       ssems[b])

    def swait(b, s):
      pltpu.make_async_copy(rows[b],
                            out_hbm.at[pl.ds(wbase + s * _SEQ, _SEQ)],
                            ssems[b]).wait()

    for b in range(_NBUF):
      pstart(b)
    for b in range(_NBUF):
      pwait(b)
      gstart(b, b)

    def body(t, carry):
      for b in range(_NBUF):
        gwait(b, t * _NBUF + b)
        sstart(b, t * _NBUF + b)
      for b in range(_NBUF):
        swait(b, t * _NBUF + b)
        pstart(b)
      for b in range(_NBUF):
        pwait(b)
        gstart(b, (t + 1) * _NBUF + b)
      return carry

    lax.fori_loop(0, _STEPS // _NBUF - 1, body, 0)

    for b in range(_NBUF):
      s = _STEPS - _NBUF + b
      gwait(b, s)
      sstart(b, s)
    for b in range(_NBUF):
      swait(b, _STEPS - _NBUF + b)

  return k(x_flat, emb_weight, pos_emb_weight)


def kernel(x, emb_weight, pos_emb_weight):
  x_flat = x.reshape(-1).astype(jnp.int32)
  out = _sc_embed(x_flat, emb_weight, pos_emb_weight)
  return out.reshape(_BATCH, _SEQ, _HID)


# speedup vs baseline: 1.7257x; 1.7076x over previous
import functools

import jax
import jax.numpy as jnp
from jax import lax
from jax.experimental import pallas as pl
from jax.experimental.pallas import tpu as pltpu
from jax.experimental.pallas import tpu_sc as plsc

_NUM_VOCAB = 100000
_MAXLEN = 200
_HID = 64
_BATCH = 4096
_SEQ = 200

_NC = 2
_NS = 16
_NW = _NC * _NS
_TOTAL = _BATCH * _SEQ
_PER_W = _TOTAL // _NW
_STEPS = _PER_W // _SEQ
_NBUF = 4
_SUBS = ((0, 128), (128, 72))


def _sc_embed(x_flat, emb_weight, pos_emb_weight):
  mesh = plsc.VectorSubcoreMesh(core_axis_name="c", subcore_axis_name="s")

  @functools.partial(
      pl.kernel,
      out_type=jax.ShapeDtypeStruct((_TOTAL, _HID), jnp.float32),
      mesh=mesh,
      compiler_params=pltpu.CompilerParams(use_tc_tiling_on_sc=False),
      scratch_types=(
          [pltpu.VMEM((_PER_W,), jnp.int32)]
          + [pltpu.VMEM_SHARED((_SEQ, _HID), jnp.float32)]
          + [pltpu.VMEM((_SEQ, _HID), jnp.float32)] * _NBUF
          + [pltpu.SemaphoreType.DMA] * _NBUF
          + [pltpu.SemaphoreType.DMA] * _NBUF
      ),
  )
  def k(x_hbm, emb_hbm, pos_hbm, out_hbm, idx_v, pos_v, *bufs_and_sems):
    rows = bufs_and_sems[:_NBUF]
    gsems = bufs_and_sems[_NBUF:2 * _NBUF]
    ssems = bufs_and_sems[2 * _NBUF:]
    wid = lax.axis_index("s") * _NC + lax.axis_index("c")
    wbase = wid * _PER_W
    pltpu.sync_copy(x_hbm.at[pl.ds(wbase, _PER_W)], idx_v)
    @pl.when(lax.axis_index("s") == 0)
    def _():
      pltpu.sync_copy(pos_hbm, pos_v)
    plsc.subcore_barrier()

    def prefill(b):
      pltpu.sync_copy(pos_v, rows[b])

    def gstart(b, s):
      row0 = s * _SEQ
      for off, n in _SUBS:
        pltpu.async_copy(
            emb_hbm.at[idx_v.at[pl.ds(row0 + off, n)]],
            rows[b].at[pl.ds(off, n)],
            gsems[b],
            add=True,
        )

    def gwait(b, s):
      row0 = s * _SEQ
      for off, n in _SUBS:
        pltpu.make_async_copy(
            emb_hbm.at[idx_v.at[pl.ds(row0 + off, n)]],
            rows[b].at[pl.ds(off, n)],
            gsems[b],
        ).wait()

    def sstart(b, s):
      pltpu.async_copy(rows[b], out_hbm.at[pl.ds(wbase + s * _SEQ, _SEQ)],
                       ssems[b])

    def swait(b, s):
      pltpu.make_async_copy(rows[b],
                            out_hbm.at[pl.ds(wbase + s * _SEQ, _SEQ)],
                            ssems[b]).wait()

    for b in range(_NBUF):
      prefill(b)
      gstart(b, b)

    def body(t, carry):
      for b in range(_NBUF):
        gwait(b, t * _NBUF + b)
        sstart(b, t * _NBUF + b)
      for b in range(_NBUF):
        swait(b, t * _NBUF + b)
        prefill(b)
        gstart(b, (t + 1) * _NBUF + b)
      return carry

    lax.fori_loop(0, _STEPS // _NBUF - 1, body, 0)

    for b in range(_NBUF):
      s = _STEPS - _NBUF + b
      gwait(b, s)
      sstart(b, s)
    for b in range(_NBUF):
      swait(b, _STEPS - _NBUF + b)

  return k(x_flat, emb_weight, pos_emb_weight)


def kernel(x, emb_weight, pos_emb_weight):
  x_flat = x.reshape(-1).astype(jnp.int32)
  out = _sc_embed(x_flat, emb_weight, pos_emb_weight)
  return out.reshape(_BATCH, _SEQ, _HID)


# speedup vs baseline: 2.2713x; 1.3162x over previous
import functools

import jax
import jax.numpy as jnp
from jax import lax
from jax.experimental import pallas as pl
from jax.experimental.pallas import tpu as pltpu
from jax.experimental.pallas import tpu_sc as plsc

_NUM_VOCAB = 100000
_MAXLEN = 200
_HID = 64
_PAD = 128
_BATCH = 4096
_SEQ = 200

_NC = 2
_NS = 16
_NW = _NC * _NS
_TOTAL = _BATCH * _SEQ
_PER_W = _TOTAL // _NW
_ROWS_W = _BATCH // _NW
_NBUF = 4
_SUBS = ((0, 128), (128, 72))


def _sc_embed(x_flat, emb128, pos128):
  mesh = plsc.VectorSubcoreMesh(core_axis_name="c", subcore_axis_name="s")

  @functools.partial(
      pl.kernel,
      out_type=jax.ShapeDtypeStruct((_BATCH, _SEQ, _PAD), jnp.float32),
      mesh=mesh,
      compiler_params=pltpu.CompilerParams(use_tc_tiling_on_sc=True),
      scratch_types=(
          [pltpu.VMEM((_PER_W,), jnp.int32)]
          + [pltpu.VMEM_SHARED((_SEQ, _PAD), jnp.float32)]
          + [pltpu.VMEM((_SEQ, _PAD), jnp.float32)] * _NBUF
          + [pltpu.SemaphoreType.DMA] * _NBUF
          + [pltpu.SemaphoreType.DMA] * _NBUF
      ),
  )
  def k(x_hbm, emb_hbm, pos_hbm, out_hbm, idx_v, pos_v, *bufs_and_sems):
    rows = bufs_and_sems[:_NBUF]
    gsems = bufs_and_sems[_NBUF:2 * _NBUF]
    ssems = bufs_and_sems[2 * _NBUF:]
    wid = lax.axis_index("s") * _NC + lax.axis_index("c")
    wrow = wid * _ROWS_W
    pltpu.sync_copy(x_hbm.at[pl.ds(wrow * _SEQ, _PER_W)], idx_v)
    @pl.when(lax.axis_index("s") == 0)
    def _():
      pltpu.sync_copy(pos_hbm, pos_v)
    plsc.subcore_barrier()

    def prefill(b):
      pltpu.sync_copy(pos_v, rows[b])

    def gstart(b, s):
      row0 = s * _SEQ
      for off, n in _SUBS:
        pltpu.async_copy(
            emb_hbm.at[idx_v.at[pl.ds(row0 + off, n)]],
            rows[b].at[pl.ds(off, n)],
            gsems[b],
            add=True,
        )

    def gwait(b, s):
      row0 = s * _SEQ
      for off, n in _SUBS:
        pltpu.make_async_copy(
            emb_hbm.at[idx_v.at[pl.ds(row0 + off, n)]],
            rows[b].at[pl.ds(off, n)],
            gsems[b],
        ).wait()

    def sstart(b, s):
      pltpu.async_copy(rows[b], out_hbm.at[wrow + s], ssems[b])

    def swait(b, s):
      pltpu.make_async_copy(rows[b], out_hbm.at[wrow + s],
                            ssems[b]).wait()

    for b in range(_NBUF):
      prefill(b)
      gstart(b, b)

    def body(t, carry):
      for b in range(_NBUF):
        gwait(b, t * _NBUF + b)
        sstart(b, t * _NBUF + b)
      for b in range(_NBUF):
        swait(b, t * _NBUF + b)
        prefill(b)
        gstart(b, (t + 1) * _NBUF + b)
      return carry

    lax.fori_loop(0, _ROWS_W // _NBUF - 1, body, 0)

    for b in range(_NBUF):
      s = _ROWS_W - _NBUF + b
      gwait(b, s)
      sstart(b, s)
    for b in range(_NBUF):
      swait(b, _ROWS_W - _NBUF + b)

  return k(x_flat, emb128, pos128)


def kernel(x, emb_weight, pos_emb_weight):
  x_flat = x.reshape(-1).astype(jnp.int32)
  emb128 = jnp.pad(emb_weight, ((0, 0), (0, _PAD - _HID)))
  pos128 = jnp.pad(pos_emb_weight, ((0, 0), (0, _PAD - _HID)))
  out = _sc_embed(x_flat, emb128, pos128)
  return out[:, :, :_HID]
